# baseline (speedup 1.0000x reference)
import functools

import jax
import jax.numpy as jnp
from jax import lax
from jax.experimental import pallas as pl
from jax.experimental.pallas import tpu as pltpu
from jax.experimental.pallas import tpu_sc as plsc

B = 16384
EMB = 32
NC = 2
NS = 16
NW = NC * NS
BPW = B // NW
CHUNK = 128
NCHUNK = BPW // CHUNK

TABLE_OF = (0, 1, 2, 2, 2, 2, 3, 3, 3)
CONCAT_ORDER = (0, 1, 2, 6, 7, 8, 3, 4, 5)


L = 16
SMALL_SLOTS = ((0, 0), (2, 1), (3, 1), (4, 1), (5, 1))
BIG_SLOTS = ((1, 0), (6, 1), (7, 1), (8, 1))


def _vec_op(dst, src, fn):
  r, c = dst.shape
  for i in range(r):
    for j in range(c // L):
      s = pl.ds(j * L, L)
      dst[i, s] = fn(src[i, s])


def _sc_gather_body(user_t, cate_t, item_q, hist_q, idx_hbm, out_hbm,
                    idx_v, gidx_v, qrem_v, rows_v, grp_v, sel_v, sem):
  c = lax.axis_index("c")
  s = lax.axis_index("s")
  wid = s * NC + c
  base = wid * BPW
  iota = lax.iota(jnp.int32, L)

  for slot, t in SMALL_SLOTS:
    tab = (user_t, cate_t)[t]
    pltpu.sync_copy(idx_hbm.at[slot * NW + wid], idx_v)
    cps = [
        pltpu.async_copy(tab.at[idx_v.at[j]],
                         rows_v.at[pl.ds(j * CHUNK, CHUNK)], sem)
        for j in range(NCHUNK)
    ]
    for cp in cps:
      cp.wait()
    pltpu.sync_copy(rows_v, out_hbm.at[slot, pl.ds(base, BPW)])

  for slot, t in BIG_SLOTS:
    tab = (item_q, hist_q)[t]
    pltpu.sync_copy(idx_hbm.at[slot * NW + wid], idx_v)
    _vec_op(gidx_v, idx_v, lambda v: v >> 2)
    _vec_op(qrem_v, idx_v, lambda v: (v & 3) * EMB)

    def chunk_body(ci, _, tab=tab, slot=slot):
      pltpu.async_copy(tab.at[gidx_v.at[ci]], grp_v, sem).wait()
      for r0 in range(0, CHUNK, L):
        rows = iota + r0
        qoff = qrem_v[ci, pl.ds(r0, L)]
        for j in range(EMB):
          vals = plsc.load_gather(grp_v, [rows, qoff + j])
          plsc.store_scatter(sel_v, [rows, jnp.full((L,), j, jnp.int32)],
                             vals)
      pltpu.sync_copy(
          sel_v, out_hbm.at[slot, pl.ds(base + ci * CHUNK, CHUNK)])
      return 0

    lax.fori_loop(0, NCHUNK, chunk_body, 0)


@functools.partial(jax.jit, static_argnames=())
def _sc_gather(user_emb, cate_emb, item_q, hist_q, idx9):
  mesh = plsc.VectorSubcoreMesh(core_axis_name="c", subcore_axis_name="s")
  k = pl.kernel(
      _sc_gather_body,
      out_type=jax.ShapeDtypeStruct((9, B, EMB), jnp.float32),
      mesh=mesh,
      scratch_types=[
          pltpu.VMEM((NCHUNK, CHUNK), jnp.int32),
          pltpu.VMEM((NCHUNK, CHUNK), jnp.int32),
          pltpu.VMEM((NCHUNK, CHUNK), jnp.int32),
          pltpu.VMEM((BPW, EMB), jnp.float32),
          pltpu.VMEM((CHUNK, 128), jnp.float32),
          pltpu.VMEM((CHUNK, EMB), jnp.float32),
          pltpu.SemaphoreType.DMA,
      ],
      compiler_params=pltpu.CompilerParams(use_tc_tiling_on_sc=False,
                                           needs_layout_passes=False),
  )
  return k(user_emb, cate_emb, item_q, hist_q, idx9)


def _dense128(t):
  return jnp.concatenate([t[q::4] for q in range(4)], axis=1)


BC = 1024


def _mlp_body(g_ref, n4_ref, w1, b1, w2, b2, w3, b3, w4, b4, out_ref):
  parts = [g_ref[k] for k in CONCAT_ORDER]
  parts.append(n4_ref[...])
  x = jnp.concatenate(parts, axis=1)
  h = jnp.maximum(
      jnp.dot(x, w1[...], preferred_element_type=jnp.float32) + b1[...], 0.0)
  h = jnp.maximum(
      jnp.dot(h, w2[...], preferred_element_type=jnp.float32) + b2[...], 0.0)
  h = jnp.maximum(
      jnp.dot(h, w3[...], preferred_element_type=jnp.float32) + b3[...], 0.0)
  z = jnp.dot(h, w4[...], preferred_element_type=jnp.float32) + b4[...]
  out_ref[...] = 1.0 / (1.0 + jnp.exp(-z))


def _mlp(gath, n4, W1, b1, W2, b2, W3, b3, W4, b4):
  full = lambda shape: pl.BlockSpec(shape, lambda i: (0,) * len(shape))
  return pl.pallas_call(
      _mlp_body,
      grid=(B // BC,),
      in_specs=[
          pl.BlockSpec((9, BC, EMB), lambda i: (0, i, 0)),
          pl.BlockSpec((BC, 4), lambda i: (i, 0)),
          full(W1.shape), full((1, 512)),
          full(W2.shape), full((1, 256)),
          full(W3.shape), full((1, 128)),
          full(W4.shape), full((1, 1)),
      ],
      out_specs=pl.BlockSpec((BC, 1), lambda i: (i, 0)),
      out_shape=jax.ShapeDtypeStruct((B, 1), jnp.float32),
  )(gath, n4, W1, b1, W2, b2, W3, b3, W4, b4)


def kernel(u, i, c, i1, i2, i3, c1, c2, c3, nv, nf, nc, nb,
           user_emb, item_emb, cate_emb, hist_emb,
           W1, b1, W2, b2, W3, b3, W4, b4):
  item_q = _dense128(item_emb)
  hist_q = _dense128(hist_emb)
  idx9 = jnp.stack([u, i, c, c1, c2, c3, i1, i2, i3]).astype(jnp.int32)
  idx9 = idx9.reshape(9 * NW, NCHUNK, CHUNK)
  gath = _sc_gather(user_emb, cate_emb, item_q, hist_q, idx9)
  n4 = jnp.stack([nv, nf, nc, nb], axis=1)
  out = _mlp(gath, n4,
             W1, b1.reshape(1, -1), W2, b2.reshape(1, -1),
             W3, b3.reshape(1, -1), W4, b4.reshape(1, -1))
  return out[:, 0]


# speedup vs baseline: 6.4887x; 6.4887x over previous
import functools

import jax
import jax.numpy as jnp
from jax import lax
from jax.experimental import pallas as pl
from jax.experimental.pallas import tpu as pltpu
from jax.experimental.pallas import tpu_sc as plsc

B = 16384
EMB = 32
NC = 2
NS = 16
NW = NC * NS
BPW = B // NW
CHUNK = 128
NCHUNK = BPW // CHUNK
N_BIG = 1000000

TABLE_OF = (0, 1, 2, 2, 2, 2, 3, 3, 3)
CONCAT_ORDER = (0, 1, 2, 6, 7, 8, 3, 4, 5)


L = 16
SMALL_SLOTS = ((0, 0), (2, 1), (3, 1), (4, 1), (5, 1))
BIG_SLOTS = ((1, 0), (6, 1), (7, 1), (8, 1))


def _vec_op(dst, src, fn):
  r, c = dst.shape
  for i in range(r):
    for j in range(c // L):
      s = pl.ds(j * L, L)
      dst[i, s] = fn(src[i, s])


def _sc_gather_body(user_t, cate_t, item_q, hist_q, idx_hbm, out_hbm,
                    idx_v, gidx_v, qrem_v, rows_v, grp_v, sel_v, sem):
  c = lax.axis_index("c")
  s = lax.axis_index("s")
  wid = s * NC + c
  base = wid * BPW
  iota = lax.iota(jnp.int32, L)

  for slot, t in SMALL_SLOTS:
    tab = (user_t, cate_t)[t]
    pltpu.sync_copy(idx_hbm.at[slot * NW + wid], idx_v)
    cps = [
        pltpu.async_copy(tab.at[idx_v.at[j]],
                         rows_v.at[pl.ds(j * CHUNK, CHUNK)], sem)
        for j in range(NCHUNK)
    ]
    for cp in cps:
      cp.wait()
    pltpu.sync_copy(rows_v, out_hbm.at[slot, pl.ds(base, BPW)])

  n4 = jnp.int32(N_BIG // 4)
  one = jnp.int32(1)
  zero = jnp.int32(0)
  for slot, t in BIG_SLOTS:
    tab = (item_q, hist_q)[t]
    pltpu.sync_copy(idx_hbm.at[slot * NW + wid], idx_v)
    for ii in range(NCHUNK):
      for jj in range(CHUNK // L):
        sl = pl.ds(jj * L, L)
        v = idx_v[ii, sl]
        q = (jnp.where(v >= n4, one, zero)
             + jnp.where(v >= 2 * n4, one, zero)
             + jnp.where(v >= 3 * n4, one, zero))
        gidx_v[ii, sl] = v - q * n4
        qrem_v[ii, sl] = q * EMB

    def chunk_body(ci, _, tab=tab, slot=slot):
      pltpu.async_copy(tab.at[gidx_v.at[ci]], grp_v, sem).wait()
      for r0 in range(0, CHUNK, L):
        rows = iota + r0
        qoff = qrem_v[ci, pl.ds(r0, L)]
        for j in range(EMB):
          vals = plsc.load_gather(grp_v, [rows, qoff + j])
          plsc.store_scatter(sel_v, [rows, jnp.full((L,), j, jnp.int32)],
                             vals)
      pltpu.sync_copy(
          sel_v, out_hbm.at[slot, pl.ds(base + ci * CHUNK, CHUNK)])
      return 0

    lax.fori_loop(0, NCHUNK, chunk_body, 0)


@functools.partial(jax.jit, static_argnames=())
def _sc_gather(user_emb, cate_emb, item_q, hist_q, idx9):
  mesh = plsc.VectorSubcoreMesh(core_axis_name="c", subcore_axis_name="s")
  k = pl.kernel(
      _sc_gather_body,
      out_type=jax.ShapeDtypeStruct((9, B, EMB), jnp.float32),
      mesh=mesh,
      scratch_types=[
          pltpu.VMEM((NCHUNK, CHUNK), jnp.int32),
          pltpu.VMEM((NCHUNK, CHUNK), jnp.int32),
          pltpu.VMEM((NCHUNK, CHUNK), jnp.int32),
          pltpu.VMEM((BPW, EMB), jnp.float32),
          pltpu.VMEM((CHUNK, 128), jnp.float32),
          pltpu.VMEM((CHUNK, EMB), jnp.float32),
          pltpu.SemaphoreType.DMA,
      ],
      compiler_params=pltpu.CompilerParams(use_tc_tiling_on_sc=False,
                                           needs_layout_passes=False),
  )
  return k(user_emb, cate_emb, item_q, hist_q, idx9)


def _dense128(t):
  n4 = t.shape[0] // 4
  return jnp.concatenate([t[q * n4:(q + 1) * n4] for q in range(4)], axis=1)


BC = 1024


def _mlp_body(g_ref, n4_ref, w1, b1, w2, b2, w3, b3, w4, b4, out_ref):
  parts = [g_ref[k] for k in CONCAT_ORDER]
  parts.append(n4_ref[...])
  x = jnp.concatenate(parts, axis=1)
  h = jnp.maximum(
      jnp.dot(x, w1[...], preferred_element_type=jnp.float32) + b1[...], 0.0)
  h = jnp.maximum(
      jnp.dot(h, w2[...], preferred_element_type=jnp.float32) + b2[...], 0.0)
  h = jnp.maximum(
      jnp.dot(h, w3[...], preferred_element_type=jnp.float32) + b3[...], 0.0)
  z = jnp.dot(h, w4[...], preferred_element_type=jnp.float32) + b4[...]
  out_ref[...] = 1.0 / (1.0 + jnp.exp(-z))


def _mlp(gath, n4, W1, b1, W2, b2, W3, b3, W4, b4):
  full = lambda shape: pl.BlockSpec(shape, lambda i: (0,) * len(shape))
  return pl.pallas_call(
      _mlp_body,
      grid=(B // BC,),
      in_specs=[
          pl.BlockSpec((9, BC, EMB), lambda i: (0, i, 0)),
          pl.BlockSpec((BC, 4), lambda i: (i, 0)),
          full(W1.shape), full((1, 512)),
          full(W2.shape), full((1, 256)),
          full(W3.shape), full((1, 128)),
          full(W4.shape), full((1, 1)),
      ],
      out_specs=pl.BlockSpec((BC, 1), lambda i: (i, 0)),
      out_shape=jax.ShapeDtypeStruct((B, 1), jnp.float32),
  )(gath, n4, W1, b1, W2, b2, W3, b3, W4, b4)


def kernel(u, i, c, i1, i2, i3, c1, c2, c3, nv, nf, nc, nb,
           user_emb, item_emb, cate_emb, hist_emb,
           W1, b1, W2, b2, W3, b3, W4, b4):
  item_q = _dense128(item_emb)
  hist_q = _dense128(hist_emb)
  idx9 = jnp.stack([u, i, c, c1, c2, c3, i1, i2, i3]).astype(jnp.int32)
  idx9 = idx9.reshape(9 * NW, NCHUNK, CHUNK)
  gath = _sc_gather(user_emb, cate_emb, item_q, hist_q, idx9)
  n4 = jnp.stack([nv, nf, nc, nb], axis=1)
  out = _mlp(gath, n4,
             W1, b1.reshape(1, -1), W2, b2.reshape(1, -1),
             W3, b3.reshape(1, -1), W4, b4.reshape(1, -1))
  return out[:, 0]


# speedup vs baseline: 10.3368x; 1.5931x over previous
import functools

import jax
import jax.numpy as jnp
from jax import lax
from jax.experimental import pallas as pl
from jax.experimental.pallas import tpu as pltpu
from jax.experimental.pallas import tpu_sc as plsc

B = 16384
EMB = 32
NC = 2
NS = 16
NW = NC * NS
BPW = B // NW
CHUNK = 128
NCHUNK = BPW // CHUNK
RING = 16
L = 16

TABLE_OF = (0, 1, 2, 2, 2, 2, 3, 3, 3)
CONCAT_ORDER = (0, 1, 2, 6, 7, 8, 3, 4, 5)


def _sc_gather_body(user_g, item_g, cate_g, hist_g, idx_hbm, out_hbm,
                    idx_v, grp_v, sel_v, idx_s, sem):
  c = lax.axis_index("c")
  s = lax.axis_index("s")
  wid = s * NC + c
  base = wid * BPW
  tables = (user_g, item_g, cate_g, hist_g)
  for slot in range(9):
    tab = tables[TABLE_OF[slot]]
    pltpu.sync_copy(idx_hbm.at[slot * NW + wid], idx_s)

    def chunk_body(ci, _, tab=tab, slot=slot):
      def batch_body(rb, _2, tab=tab):
        idx16 = idx_s[ci, pl.ds(rb * RING, RING)]
        gi16 = idx16 >> 3
        sub16 = idx16 & 7
        for rr in range(RING):
          pltpu.async_copy(tab.at[gi16[rr]], grp_v.at[rr], sem)
        for rr in range(RING):
          pltpu.make_async_copy(tab.at[0], grp_v.at[0], sem).wait()
        for rr in range(RING):
          sv = sub16[rr]
          r = rb * RING + rr
          sel_v[r, pl.ds(0, L)] = grp_v[rr, sv, pl.ds(0, L)]
          sel_v[r, pl.ds(L, L)] = grp_v[rr, sv, pl.ds(L, L)]
        return 0

      lax.fori_loop(0, CHUNK // RING, batch_body, 0)
      pltpu.sync_copy(sel_v,
                      out_hbm.at[slot, pl.ds(base + ci * CHUNK, CHUNK)])
      return 0

    lax.fori_loop(0, NCHUNK, chunk_body, 0)


@functools.partial(jax.jit, static_argnames=())
def _sc_gather(user_emb, item_emb, cate_emb, hist_emb, idx9):
  mesh = plsc.VectorSubcoreMesh(core_axis_name="c", subcore_axis_name="s")
  k = pl.kernel(
      _sc_gather_body,
      out_type=jax.ShapeDtypeStruct((9, B, EMB), jnp.float32),
      mesh=mesh,
      scratch_types=[
          pltpu.VMEM((NCHUNK, CHUNK), jnp.int32),
          pltpu.VMEM((RING, 8, EMB), jnp.float32),
          pltpu.VMEM((CHUNK, EMB), jnp.float32),
          pltpu.VMEM((NCHUNK, CHUNK), jnp.int32),
          pltpu.SemaphoreType.DMA,
      ],
      compiler_params=pltpu.CompilerParams(needs_layout_passes=False),
  )
  return k(user_emb.reshape(-1, 8, EMB), item_emb.reshape(-1, 8, EMB),
           cate_emb.reshape(-1, 8, EMB), hist_emb.reshape(-1, 8, EMB),
           idx9)


BC = 1024


def _mlp_body(g_ref, n4_ref, w1, b1, w2, b2, w3, b3, w4, b4, out_ref):
  parts = [g_ref[k] for k in CONCAT_ORDER]
  parts.append(n4_ref[...])
  x = jnp.concatenate(parts, axis=1)
  h = jnp.maximum(
      jnp.dot(x, w1[...], preferred_element_type=jnp.float32) + b1[...], 0.0)
  h = jnp.maximum(
      jnp.dot(h, w2[...], preferred_element_type=jnp.float32) + b2[...], 0.0)
  h = jnp.maximum(
      jnp.dot(h, w3[...], preferred_element_type=jnp.float32) + b3[...], 0.0)
  z = jnp.dot(h, w4[...], preferred_element_type=jnp.float32) + b4[...]
  out_ref[...] = 1.0 / (1.0 + jnp.exp(-z))


def _mlp(gath, n4, W1, b1, W2, b2, W3, b3, W4, b4):
  full = lambda shape: pl.BlockSpec(shape, lambda i: (0,) * len(shape))
  return pl.pallas_call(
      _mlp_body,
      grid=(B // BC,),
      in_specs=[
          pl.BlockSpec((9, BC, EMB), lambda i: (0, i, 0)),
          pl.BlockSpec((BC, 4), lambda i: (i, 0)),
          full(W1.shape), full((1, 512)),
          full(W2.shape), full((1, 256)),
          full(W3.shape), full((1, 128)),
          full(W4.shape), full((1, 1)),
      ],
      out_specs=pl.BlockSpec((BC, 1), lambda i: (i, 0)),
      out_shape=jax.ShapeDtypeStruct((B, 1), jnp.float32),
  )(gath, n4, W1, b1, W2, b2, W3, b3, W4, b4)


def kernel(u, i, c, i1, i2, i3, c1, c2, c3, nv, nf, nc, nb,
           user_emb, item_emb, cate_emb, hist_emb,
           W1, b1, W2, b2, W3, b3, W4, b4):
  idx9 = jnp.stack([u, i, c, c1, c2, c3, i1, i2, i3]).astype(jnp.int32)
  idx9 = idx9.reshape(9 * NW, NCHUNK, CHUNK)
  gath = _sc_gather(user_emb, item_emb, cate_emb, hist_emb, idx9)
  n4 = jnp.stack([nv, nf, nc, nb], axis=1)
  out = _mlp(gath, n4,
             W1, b1.reshape(1, -1), W2, b2.reshape(1, -1),
             W3, b3.reshape(1, -1), W4, b4.reshape(1, -1))
  return out[:, 0]
